# baseline (speedup 1.0000x reference)
import functools

import jax
import jax.numpy as jnp
from jax import lax
from jax.experimental import pallas as pl
from jax.experimental.pallas import tpu as pltpu
from jax.experimental.pallas import tpu_sc as plsc

_INFO = plsc.get_sparse_core_info()
_NC = _INFO.num_cores
_NS = _INFO.num_subcores
_NL = _INFO.num_lanes
_NW = _NC * _NS

_C = 80
_NB = 3


def _deg_pass(ei5, npad):
    _, _, nph, cpp, c_w = ei5.shape
    nch = nph * cpp
    rpt = npad // _NS
    mesh = plsc.VectorSubcoreMesh(core_axis_name="c", subcore_axis_name="s")

    @functools.partial(
        pl.kernel,
        out_type=(jax.ShapeDtypeStruct((npad,), jnp.float32),
                  jax.ShapeDtypeStruct((npad,), jnp.float32)),
        mesh=mesh,
        scratch_types=[
            pltpu.VMEM_SHARED((npad,), jnp.float32),
            pltpu.VMEM((nch, c_w), jnp.int32),
            pltpu.VMEM((((c_w + _NL - 1) // _NL) * _NL,), jnp.float32),
            pltpu.VMEM((((rpt + _NL - 1) // _NL) * _NL,), jnp.float32),
            pltpu.SemaphoreType.DMA,
        ],
    )
    def k(ei_hbm, d0_hbm, d1_hbm, deg_sh, cidx, ones_v, zbuf, sem):
        c = lax.axis_index("c")
        s = lax.axis_index("s")
        w = c * _NS + s
        rbase = s * rpt

        def zrow(i, _):
            zbuf[pl.ds(i * _NL, _NL)] = jnp.zeros((_NL,), jnp.float32)
            return 0
        lax.fori_loop(0, zbuf.shape[0] // _NL, zrow, 0)
        for t in range(ones_v.shape[0] // _NL):
            ones_v[pl.ds(t * _NL, _NL)] = jnp.ones((_NL,), jnp.float32)
        pltpu.sync_copy(zbuf.at[pl.ds(0, rpt)], deg_sh.at[pl.ds(rbase, rpt)])
        for p in range(nph):
            pltpu.sync_copy(ei_hbm.at[1, w, p],
                            cidx.at[pl.ds(p * cpp, cpp)])
        plsc.subcore_barrier()

        def fire(k_, _):
            pltpu.async_copy(ones_v.at[pl.ds(0, c_w)],
                             deg_sh.at[cidx.at[k_]], sem, add=True)
            return 0
        lax.fori_loop(0, nch, fire, 0)

        def drain(k_, _):
            pltpu.make_async_copy(ones_v.at[pl.ds(0, c_w)],
                                  deg_sh.at[cidx.at[0]], sem).wait()
            return 0
        lax.fori_loop(0, nch, drain, 0)
        plsc.subcore_barrier()

        @pl.when(c == 0)
        def _():
            pltpu.sync_copy(deg_sh.at[pl.ds(rbase, rpt)],
                            d0_hbm.at[pl.ds(rbase, rpt)])

        @pl.when(c != 0)
        def _():
            pltpu.sync_copy(deg_sh.at[pl.ds(rbase, rpt)],
                            d1_hbm.at[pl.ds(rbase, rpt)])

    return k(ei5)


def _edge_pass(y, ei5):
    npad, d = y.shape
    _, _, nph, cpp, c_w = ei5.shape
    assert cpp > _NB
    ngrp = cpp // _NB
    ntail = cpp - ngrp * _NB
    rpt = npad // _NS
    assert rpt % 8 == 0
    mesh = plsc.VectorSubcoreMesh(core_axis_name="c", subcore_axis_name="s")

    @functools.partial(
        pl.kernel,
        out_type=(jax.ShapeDtypeStruct((npad, d), jnp.float32),
                  jax.ShapeDtypeStruct((npad, d), jnp.float32)),
        mesh=mesh,
        scratch_types=(
            [pltpu.VMEM_SHARED((npad, d), jnp.float32),
             pltpu.VMEM((2, cpp, c_w), jnp.int32),
             pltpu.VMEM((2, cpp, c_w), jnp.int32)]
            + [pltpu.VMEM((c_w, d), jnp.float32)] * _NB
            + [pltpu.SemaphoreType.DMA] * (2 * _NB + 1)
        ),
    )
    def k(y_hbm, ei_hbm, p0_hbm, p1_hbm,
          acc_sh, ridx, cidx, *bufs_and_sems):
        m = bufs_and_sems[:_NB]
        gs = bufs_and_sems[_NB:2 * _NB]
        ss = bufs_and_sems[2 * _NB:3 * _NB]
        si = bufs_and_sems[3 * _NB]
        c = lax.axis_index("c")
        s = lax.axis_index("s")
        w = c * _NS + s
        rbase = s * rpt

        @pl.when(c == 0)
        def _():
            pltpu.sync_copy(y_hbm.at[pl.ds(rbase, rpt)],
                            acc_sh.at[pl.ds(rbase, rpt)])

        @pl.when(c != 0)
        def _():
            def zrow(i, _):
                for jj in range(d // _NL):
                    m[0][i, pl.ds(jj * _NL, _NL)] = jnp.zeros((_NL,), jnp.float32)
                return 0
            lax.fori_loop(0, c_w, zrow, 0)
            zc = (c_w // 8) * 8
            nfull = rpt // zc
            for t in range(nfull):
                pltpu.sync_copy(m[0].at[pl.ds(0, zc)],
                                acc_sh.at[pl.ds(rbase + t * zc, zc)])
            rem = rpt - nfull * zc
            if rem:
                pltpu.sync_copy(
                    m[0].at[pl.ds(0, rem)],
                    acc_sh.at[pl.ds(rbase + nfull * zc, rem)])

        plsc.subcore_barrier()

        def start(sl, k_, b):
            pltpu.async_copy(y_hbm.at[ridx.at[sl, k_]], m[b], gs[b])

        def wait_g(b):
            pltpu.make_async_copy(y_hbm.at[ridx.at[0, 0]], m[b], gs[b]).wait()

        def scat(sl, k_, b):
            pltpu.async_copy(m[b], acc_sh.at[cidx.at[sl, k_]], ss[b], add=True)

        def wait_s(b):
            pltpu.make_async_copy(m[b], acc_sh.at[cidx.at[0, 0]], ss[b]).wait()

        pltpu.sync_copy(ei_hbm.at[0, w, 0], ridx.at[0])
        pltpu.sync_copy(ei_hbm.at[1, w, 0], cidx.at[0])

        def phase(p, _):
            sl = lax.rem(p, 2)

            @pl.when(p > 0)
            def _():
                wait_g_idx = pltpu.make_async_copy(
                    ei_hbm.at[0, w, 0], ridx.at[0], si)
                wait_g_idx.wait()
                wait_g_idx.wait()

            @pl.when(p + 1 < nph)
            def _():
                pltpu.async_copy(ei_hbm.at[0, w, p + 1], ridx.at[1 - sl], si)
                pltpu.async_copy(ei_hbm.at[1, w, p + 1], cidx.at[1 - sl], si)

            for b in range(_NB):
                start(sl, b, b)

            def group(g, _):
                for b in range(_NB):
                    k_ = g * _NB + b
                    wait_g(b)
                    scat(sl, k_, b)
                for b in range(_NB):
                    kn = (g + 1) * _NB + b

                    @pl.when(kn < cpp)
                    def _():
                        wait_s(b)
                        start(sl, kn, b)
                return 0

            lax.fori_loop(0, ngrp, group, 0)
            for b in range(ntail):
                k_ = ngrp * _NB + b
                wait_g(b)
                scat(sl, k_, b)
            for b in range(_NB):
                wait_s(b)
            return 0

        lax.fori_loop(0, nph, phase, 0)
        plsc.subcore_barrier()

        @pl.when(c == 0)
        def _():
            pltpu.sync_copy(acc_sh.at[pl.ds(rbase, rpt)],
                            p0_hbm.at[pl.ds(rbase, rpt)])

        @pl.when(c != 0)
        def _():
            pltpu.sync_copy(acc_sh.at[pl.ds(rbase, rpt)],
                            p1_hbm.at[pl.ds(rbase, rpt)])

    return k(y, ei5)


_NROWBLK = 8


def _t1_body(x_ref, w_ref, dg_ref, o_ref):
    dis = lax.rsqrt(dg_ref[...] + 1.0)
    o_ref[...] = dis * jnp.dot(x_ref[...], w_ref[...],
                               preferred_element_type=jnp.float32)


def _t1(x, w, dg, npad):
    _, d_in = x.shape
    d_out = w.shape[1]
    bm = npad // _NROWBLK
    return pl.pallas_call(
        _t1_body,
        out_shape=jax.ShapeDtypeStruct((npad, d_out), jnp.float32),
        grid=(_NROWBLK,),
        in_specs=[
            pl.BlockSpec((bm, d_in), lambda i: (i, 0)),
            pl.BlockSpec((d_in, d_out), lambda i: (0, 0)),
            pl.BlockSpec((bm, 1), lambda i: (i, 0)),
        ],
        out_specs=pl.BlockSpec((bm, d_out), lambda i: (i, 0)),
    )(x, w, dg)


def _t2_body(p0_ref, p1_ref, dg_ref, b_ref, w_ref, o_ref):
    dis = lax.rsqrt(dg_ref[...] + 1.0)
    h = jnp.maximum(dis * (p0_ref[...] + p1_ref[...]) + b_ref[...], 0.0)
    o_ref[...] = dis * jnp.dot(h, w_ref[...],
                               preferred_element_type=jnp.float32)


def _t2(p0, p1, dg, b, w):
    npad, d = p0.shape
    d_out = w.shape[1]
    bm = npad // _NROWBLK
    return pl.pallas_call(
        _t2_body,
        out_shape=jax.ShapeDtypeStruct((npad, d_out), jnp.float32),
        grid=(_NROWBLK,),
        in_specs=[
            pl.BlockSpec((bm, d), lambda i: (i, 0)),
            pl.BlockSpec((bm, d), lambda i: (i, 0)),
            pl.BlockSpec((bm, 1), lambda i: (i, 0)),
            pl.BlockSpec((1, d), lambda i: (0, 0)),
            pl.BlockSpec((d, d_out), lambda i: (0, 0)),
        ],
        out_specs=pl.BlockSpec((bm, d_out), lambda i: (i, 0)),
    )(p0, p1, dg, b, w)


def _t3_body(q0_ref, q1_ref, dg_ref, b_ref, o_ref):
    dis = lax.rsqrt(dg_ref[...] + 1.0)
    o_ref[...] = dis * (q0_ref[...] + q1_ref[...]) + b_ref[...]


def _t3(q0, q1, dg, b, n):
    npad, d = q0.shape
    bm = npad // _NROWBLK
    return pl.pallas_call(
        _t3_body,
        out_shape=jax.ShapeDtypeStruct((n, d), jnp.float32),
        grid=(_NROWBLK,),
        in_specs=[
            pl.BlockSpec((bm, d), lambda i: (i, 0)),
            pl.BlockSpec((bm, d), lambda i: (i, 0)),
            pl.BlockSpec((bm, 1), lambda i: (i, 0)),
            pl.BlockSpec((1, d), lambda i: (0, 0)),
        ],
        out_specs=pl.BlockSpec((bm, d), lambda i: (i, 0)),
    )(q0, q1, dg, b)


def kernel(x, edge_index, W0, b0, W1, b1):
    n, _ = x.shape
    e = edge_index.shape[1]
    assert e % (_NW * _C) == 0, (e, _NW, _C)
    grp = _NS * 8
    npad = ((n + grp - 1) // grp) * grp
    assert npad % (_NROWBLK * 8) == 0

    nch = e // (_NW * _C)
    nph = 5
    assert nch % nph == 0 and nch // nph > _NB
    ei5 = edge_index.reshape(2, _NW, nph, nch // nph, _C)

    deg_grp = _NS * 128
    npad_deg = ((n + deg_grp - 1) // deg_grp) * deg_grp
    d0, d1 = _deg_pass(ei5, npad_deg)
    dg = (d0 + d1)[:npad, None]

    y0 = _t1(x, W0, dg, npad)
    p0, p1 = _edge_pass(y0, ei5)
    y1 = _t2(p0, p1, dg, b0[None, :], W1)
    q0, q1 = _edge_pass(y1, ei5)
    return _t3(q0, q1, dg, b1[None, :], n)


# speedup vs baseline: 1.0684x; 1.0684x over previous
import functools

import jax
import jax.numpy as jnp
from jax import lax
from jax.experimental import pallas as pl
from jax.experimental.pallas import tpu as pltpu
from jax.experimental.pallas import tpu_sc as plsc

_INFO = plsc.get_sparse_core_info()
_NC = _INFO.num_cores
_NS = _INFO.num_subcores
_NL = _INFO.num_lanes
_NW = _NC * _NS

_C = 80
_NB = 4


def _deg_pass(ei5, npad):
    _, _, nph, cpp, c_w = ei5.shape
    nch = nph * cpp
    rpt = npad // _NS
    mesh = plsc.VectorSubcoreMesh(core_axis_name="c", subcore_axis_name="s")

    @functools.partial(
        pl.kernel,
        out_type=(jax.ShapeDtypeStruct((npad,), jnp.float32),
                  jax.ShapeDtypeStruct((npad,), jnp.float32)),
        mesh=mesh,
        scratch_types=[
            pltpu.VMEM_SHARED((npad,), jnp.float32),
            pltpu.VMEM((nch, c_w), jnp.int32),
            pltpu.VMEM((((c_w + _NL - 1) // _NL) * _NL,), jnp.float32),
            pltpu.VMEM((((rpt + _NL - 1) // _NL) * _NL,), jnp.float32),
            pltpu.SemaphoreType.DMA,
        ],
    )
    def k(ei_hbm, d0_hbm, d1_hbm, deg_sh, cidx, ones_v, zbuf, sem):
        c = lax.axis_index("c")
        s = lax.axis_index("s")
        w = c * _NS + s
        rbase = s * rpt

        def zrow(i, _):
            zbuf[pl.ds(i * _NL, _NL)] = jnp.zeros((_NL,), jnp.float32)
            return 0
        lax.fori_loop(0, zbuf.shape[0] // _NL, zrow, 0)
        for t in range(ones_v.shape[0] // _NL):
            ones_v[pl.ds(t * _NL, _NL)] = jnp.ones((_NL,), jnp.float32)
        pltpu.sync_copy(zbuf.at[pl.ds(0, rpt)], deg_sh.at[pl.ds(rbase, rpt)])
        for p in range(nph):
            pltpu.sync_copy(ei_hbm.at[1, w, p],
                            cidx.at[pl.ds(p * cpp, cpp)])
        plsc.subcore_barrier()

        def fire(k_, _):
            pltpu.async_copy(ones_v.at[pl.ds(0, c_w)],
                             deg_sh.at[cidx.at[k_]], sem, add=True)
            return 0
        lax.fori_loop(0, nch, fire, 0)

        def drain(k_, _):
            pltpu.make_async_copy(ones_v.at[pl.ds(0, c_w)],
                                  deg_sh.at[cidx.at[0]], sem).wait()
            return 0
        lax.fori_loop(0, nch, drain, 0)
        plsc.subcore_barrier()

        @pl.when(c == 0)
        def _():
            pltpu.sync_copy(deg_sh.at[pl.ds(rbase, rpt)],
                            d0_hbm.at[pl.ds(rbase, rpt)])

        @pl.when(c != 0)
        def _():
            pltpu.sync_copy(deg_sh.at[pl.ds(rbase, rpt)],
                            d1_hbm.at[pl.ds(rbase, rpt)])

    return k(ei5)


def _edge_pass(y, ei5):
    npad, d = y.shape
    _, _, nph, cpp, c_w = ei5.shape
    assert cpp > _NB
    ngrp = cpp // _NB
    ntail = cpp - ngrp * _NB
    rpt = npad // _NS
    assert rpt % 8 == 0
    mesh = plsc.VectorSubcoreMesh(core_axis_name="c", subcore_axis_name="s")

    @functools.partial(
        pl.kernel,
        out_type=(jax.ShapeDtypeStruct((npad, d), jnp.float32),
                  jax.ShapeDtypeStruct((npad, d), jnp.float32)),
        mesh=mesh,
        scratch_types=(
            [pltpu.VMEM_SHARED((npad, d), jnp.float32),
             pltpu.VMEM((cpp, c_w), jnp.int32),
             pltpu.VMEM((cpp, c_w), jnp.int32)]
            + [pltpu.VMEM((c_w, d), jnp.float32)] * _NB
            + [pltpu.SemaphoreType.DMA] * (2 * _NB + 1)
        ),
    )
    def k(y_hbm, ei_hbm, p0_hbm, p1_hbm,
          acc_sh, ridx, cidx, *bufs_and_sems):
        m = bufs_and_sems[:_NB]
        gs = bufs_and_sems[_NB:2 * _NB]
        ss = bufs_and_sems[2 * _NB:3 * _NB]
        si = bufs_and_sems[3 * _NB]
        c = lax.axis_index("c")
        s = lax.axis_index("s")
        w = c * _NS + s
        rbase = s * rpt

        @pl.when(c == 0)
        def _():
            pltpu.sync_copy(y_hbm.at[pl.ds(rbase, rpt)],
                            acc_sh.at[pl.ds(rbase, rpt)])

        @pl.when(c != 0)
        def _():
            def zrow(i, _):
                for jj in range(d // _NL):
                    m[0][i, pl.ds(jj * _NL, _NL)] = jnp.zeros((_NL,), jnp.float32)
                return 0
            lax.fori_loop(0, c_w, zrow, 0)
            zc = (c_w // 8) * 8
            nfull = rpt // zc
            for t in range(nfull):
                pltpu.sync_copy(m[0].at[pl.ds(0, zc)],
                                acc_sh.at[pl.ds(rbase + t * zc, zc)])
            rem = rpt - nfull * zc
            if rem:
                pltpu.sync_copy(
                    m[0].at[pl.ds(0, rem)],
                    acc_sh.at[pl.ds(rbase + nfull * zc, rem)])

        plsc.subcore_barrier()

        def start(k_, b):
            pltpu.async_copy(y_hbm.at[ridx.at[k_]], m[b], gs[b])

        def wait_g(b):
            pltpu.make_async_copy(y_hbm.at[ridx.at[0]], m[b], gs[b]).wait()

        def scat(k_, b):
            pltpu.async_copy(m[b], acc_sh.at[cidx.at[k_]], ss[b], add=True)

        def wait_s(b):
            pltpu.make_async_copy(m[b], acc_sh.at[cidx.at[0]], ss[b]).wait()

        pltpu.sync_copy(ei_hbm.at[0, w, 0], ridx)
        for b in range(_NB):
            start(b, b)
        pltpu.sync_copy(ei_hbm.at[1, w, 0], cidx)

        def phase(p, _):
            def group(g, _):
                for b in range(_NB):
                    k_ = g * _NB + b
                    wait_g(b)
                    scat(k_, b)
                for b in range(_NB):
                    kn = (g + 1) * _NB + b

                    @pl.when(kn < cpp)
                    def _():
                        wait_s(b)
                        start(kn, b)
                return 0

            lax.fori_loop(0, ngrp, group, 0)
            for b in range(ntail):
                k_ = ngrp * _NB + b
                wait_g(b)
                scat(k_, b)

            @pl.when(p + 1 < nph)
            def _():
                pltpu.async_copy(ei_hbm.at[0, w, p + 1], ridx, si)
            for b in range(_NB):
                wait_s(b)

            @pl.when(p + 1 < nph)
            def _():
                pltpu.make_async_copy(ei_hbm.at[0, w, 0], ridx, si).wait()
                for b in range(_NB):
                    start(b, b)
                pltpu.sync_copy(ei_hbm.at[1, w, p + 1], cidx)
            return 0

        lax.fori_loop(0, nph, phase, 0)
        plsc.subcore_barrier()

        @pl.when(c == 0)
        def _():
            pltpu.sync_copy(acc_sh.at[pl.ds(rbase, rpt)],
                            p0_hbm.at[pl.ds(rbase, rpt)])

        @pl.when(c != 0)
        def _():
            pltpu.sync_copy(acc_sh.at[pl.ds(rbase, rpt)],
                            p1_hbm.at[pl.ds(rbase, rpt)])

    return k(y, ei5)


_NROWBLK = 8


def _t1_body(x_ref, w_ref, dg_ref, o_ref):
    dis = lax.rsqrt(dg_ref[...] + 1.0)
    o_ref[...] = dis * jnp.dot(x_ref[...], w_ref[...],
                               preferred_element_type=jnp.float32)


def _t1(x, w, dg, npad):
    _, d_in = x.shape
    d_out = w.shape[1]
    bm = npad // _NROWBLK
    return pl.pallas_call(
        _t1_body,
        out_shape=jax.ShapeDtypeStruct((npad, d_out), jnp.float32),
        grid=(_NROWBLK,),
        in_specs=[
            pl.BlockSpec((bm, d_in), lambda i: (i, 0)),
            pl.BlockSpec((d_in, d_out), lambda i: (0, 0)),
            pl.BlockSpec((bm, 1), lambda i: (i, 0)),
        ],
        out_specs=pl.BlockSpec((bm, d_out), lambda i: (i, 0)),
    )(x, w, dg)


def _t2_body(p0_ref, p1_ref, dg_ref, b_ref, w_ref, o_ref):
    dis = lax.rsqrt(dg_ref[...] + 1.0)
    h = jnp.maximum(dis * (p0_ref[...] + p1_ref[...]) + b_ref[...], 0.0)
    o_ref[...] = dis * jnp.dot(h, w_ref[...],
                               preferred_element_type=jnp.float32)


def _t2(p0, p1, dg, b, w):
    npad, d = p0.shape
    d_out = w.shape[1]
    bm = npad // _NROWBLK
    return pl.pallas_call(
        _t2_body,
        out_shape=jax.ShapeDtypeStruct((npad, d_out), jnp.float32),
        grid=(_NROWBLK,),
        in_specs=[
            pl.BlockSpec((bm, d), lambda i: (i, 0)),
            pl.BlockSpec((bm, d), lambda i: (i, 0)),
            pl.BlockSpec((bm, 1), lambda i: (i, 0)),
            pl.BlockSpec((1, d), lambda i: (0, 0)),
            pl.BlockSpec((d, d_out), lambda i: (0, 0)),
        ],
        out_specs=pl.BlockSpec((bm, d_out), lambda i: (i, 0)),
    )(p0, p1, dg, b, w)


def _t3_body(q0_ref, q1_ref, dg_ref, b_ref, o_ref):
    dis = lax.rsqrt(dg_ref[...] + 1.0)
    o_ref[...] = dis * (q0_ref[...] + q1_ref[...]) + b_ref[...]


def _t3(q0, q1, dg, b, n):
    npad, d = q0.shape
    bm = npad // _NROWBLK
    return pl.pallas_call(
        _t3_body,
        out_shape=jax.ShapeDtypeStruct((n, d), jnp.float32),
        grid=(_NROWBLK,),
        in_specs=[
            pl.BlockSpec((bm, d), lambda i: (i, 0)),
            pl.BlockSpec((bm, d), lambda i: (i, 0)),
            pl.BlockSpec((bm, 1), lambda i: (i, 0)),
            pl.BlockSpec((1, d), lambda i: (0, 0)),
        ],
        out_specs=pl.BlockSpec((bm, d), lambda i: (i, 0)),
    )(q0, q1, dg, b)


def kernel(x, edge_index, W0, b0, W1, b1):
    n, _ = x.shape
    e = edge_index.shape[1]
    assert e % (_NW * _C) == 0, (e, _NW, _C)
    grp = _NS * 8
    npad = ((n + grp - 1) // grp) * grp
    assert npad % (_NROWBLK * 8) == 0

    nch = e // (_NW * _C)
    nph = 5
    assert nch % nph == 0 and nch // nph > _NB
    ei5 = edge_index.reshape(2, _NW, nph, nch // nph, _C)

    deg_grp = _NS * 128
    npad_deg = ((n + deg_grp - 1) // deg_grp) * deg_grp
    d0, d1 = _deg_pass(ei5, npad_deg)
    dg = (d0 + d1)[:npad, None]

    y0 = _t1(x, W0, dg, npad)
    p0, p1 = _edge_pass(y0, ei5)
    y1 = _t2(p0, p1, dg, b0[None, :], W1)
    q0, q1 = _edge_pass(y1, ei5)
    return _t3(q0, q1, dg, b1[None, :], n)
